# baseline (speedup 1.0000x reference)
import functools

import jax
import jax.numpy as jnp
from jax import lax
from jax.experimental import pallas as pl
from jax.experimental.pallas import tpu as pltpu
from jax.experimental.pallas import tpu_sc as plsc

EMBED = 64
LABELS = 4
B = 4096
L = 50

NC, NS, LANES = 2, 16, 16
NW = NC * NS
BPW = B // NW
SEGS = EMBED // LANES
HB = B // 2

_mesh = plsc.VectorSubcoreMesh(
    core_axis_name="c", subcore_axis_name="s", num_cores=NC, num_subcores=NS
)


LP = 56


def _widen_body(t_ref, o_ref):
    o_ref[...] = jnp.zeros((B, 128), jnp.int32)
    o_ref[:, :L] = t_ref[...]


_widen = pl.pallas_call(
    _widen_body,
    out_shape=jax.ShapeDtypeStruct((B, 128), jnp.int32),
)


def _example_mean(rows_v, slot, out_v, e):
    def red(l, acc):
        return tuple(
            acc[g] + rows_v[slot, l, pl.ds(g * LANES, LANES)]
            for g in range(SEGS)
        )
    init = tuple(jnp.zeros((LANES,), jnp.float32) for _ in range(SEGS))
    acc = lax.fori_loop(0, L, red, init, unroll=5)
    for g in range(SEGS):
        out_v[e, pl.ds(g * LANES, LANES)] = acc[g] * (1.0 / L)


@functools.partial(
    pl.kernel,
    out_type=jax.ShapeDtypeStruct((HB, 2 * EMBED), jnp.float32),
    mesh=_mesh,
    scratch_types=[
        pltpu.VMEM((BPW, 128), jnp.int32),
        pltpu.VMEM((2, LP, EMBED), jnp.float32),
        pltpu.VMEM((BPW, EMBED), jnp.float32),
        pltpu.SemaphoreType.DMA,
        pltpu.SemaphoreType.DMA,
    ],
    compiler_params=pltpu.CompilerParams(use_tc_tiling_on_sc=False),
)
def _embed_bag(text_hbm, table_hbm, out_hbm, idx_v, rows_v, out_v, sem0, sem1):
    wid = lax.axis_index("s") * NC + lax.axis_index("c")
    pltpu.sync_copy(text_hbm.at[pl.ds(wid * BPW, BPW)], idx_v)

    pltpu.async_copy(table_hbm.at[idx_v.at[0, pl.ds(0, LP)]], rows_v.at[0], sem0)

    def pair_body(pp, _):
        e = 2 * pp
        pltpu.async_copy(
            table_hbm.at[idx_v.at[e + 1, pl.ds(0, LP)]], rows_v.at[1], sem1
        )
        pltpu.make_async_copy(
            table_hbm.at[idx_v.at[e, pl.ds(0, LP)]], rows_v.at[0], sem0
        ).wait()
        _example_mean(rows_v, 0, out_v, e)

        @pl.when(e + 2 < BPW)
        def _():
            pltpu.async_copy(
                table_hbm.at[idx_v.at[e + 2, pl.ds(0, LP)]], rows_v.at[0], sem0
            )

        pltpu.make_async_copy(
            table_hbm.at[idx_v.at[e + 1, pl.ds(0, LP)]], rows_v.at[1], sem1
        ).wait()
        _example_mean(rows_v, 1, out_v, e + 1)
        return 0

    lax.fori_loop(0, BPW // 2, pair_body, 0)

    row_base = (wid & 15) * BPW
    col_base = (wid >> 4) * EMBED
    pltpu.sync_copy(
        out_v,
        out_hbm.at[pl.ds(row_base, BPW), pl.ds(col_base, EMBED)],
    )


def _head_body(emb_ref, w_ref, b_ref, out_ref):
    e = emb_ref[...]
    w = w_ref[...]
    bb = b_ref[...]
    dn = (((1,), (1,)), ((), ()))
    top = lax.dot_general(e[:, :EMBED], w, dn, preferred_element_type=jnp.float32)
    bot = lax.dot_general(e[:, EMBED:], w, dn, preferred_element_type=jnp.float32)
    out_ref[:HB, :] = 1.0 / (1.0 + jnp.exp(-(top + bb)))
    out_ref[HB:, :] = 1.0 / (1.0 + jnp.exp(-(bot + bb)))


_head = pl.pallas_call(
    _head_body,
    out_shape=jax.ShapeDtypeStruct((B, LABELS), jnp.float32),
)


def kernel(text, table, W, b):
    text_w = _widen(text)
    emb2 = _embed_bag(text_w, table)
    return _head(emb2, W, b.reshape(1, LABELS))


# speedup vs baseline: 1.5667x; 1.5667x over previous
import functools

import jax
import jax.numpy as jnp
from jax import lax
from jax.experimental import pallas as pl
from jax.experimental.pallas import tpu as pltpu
from jax.experimental.pallas import tpu_sc as plsc

EMBED = 64
LABELS = 4
B = 4096
L = 50

NC, NS, LANES = 2, 16, 16
NW = NC * NS
BPW = B // NW
IPW = BPW * L
CHUNKS = IPW // 128
SEGS = EMBED // LANES
HB = B // 2

_mesh = plsc.VectorSubcoreMesh(
    core_axis_name="c", subcore_axis_name="s", num_cores=NC, num_subcores=NS
)


def _process_chunk(rows_v, slot, c, out_v):
    def row_body(i, _):
        p = c * 128 + i
        el = (p * 5243) >> 18
        for g in range(SEGS):
            plsc.addupdate(
                out_v.at[el, pl.ds(g * LANES, LANES)],
                rows_v[slot, i, pl.ds(g * LANES, LANES)],
            )
        return 0
    lax.fori_loop(0, 128, row_body, 0, unroll=4)


@functools.partial(
    pl.kernel,
    out_type=jax.ShapeDtypeStruct((HB, 2 * EMBED), jnp.float32),
    mesh=_mesh,
    scratch_types=[
        pltpu.VMEM((IPW,), jnp.int32),
        pltpu.VMEM((2, 128, EMBED), jnp.float32),
        pltpu.VMEM((BPW, EMBED), jnp.float32),
        pltpu.SemaphoreType.DMA,
        pltpu.SemaphoreType.DMA,
    ],
    compiler_params=pltpu.CompilerParams(use_tc_tiling_on_sc=False),
)
def _embed_bag(text_hbm, table_hbm, out_hbm, idx_v, rows_v, out_v, sem0, sem1):
    wid = lax.axis_index("s") * NC + lax.axis_index("c")
    pltpu.sync_copy(text_hbm.at[pl.ds(wid * IPW, IPW)], idx_v)

    zero = jnp.zeros((LANES,), jnp.float32)
    def zero_body(r, _):
        for g in range(SEGS):
            out_v[r, pl.ds(g * LANES, LANES)] = zero
        return 0
    lax.fori_loop(0, BPW, zero_body, 0, unroll=4)

    pltpu.async_copy(
        table_hbm.at[idx_v.at[pl.ds(0, 128)]], rows_v.at[0], sem0
    )

    def pair_body(pp, _):
        base = 2 * pp
        pltpu.async_copy(
            table_hbm.at[idx_v.at[pl.ds((base + 1) * 128, 128)]],
            rows_v.at[1], sem1,
        )
        pltpu.make_async_copy(
            table_hbm.at[idx_v.at[pl.ds(base * 128, 128)]], rows_v.at[0], sem0
        ).wait()
        _process_chunk(rows_v, 0, base, out_v)

        @pl.when(base + 2 < CHUNKS)
        def _():
            pltpu.async_copy(
                table_hbm.at[idx_v.at[pl.ds((base + 2) * 128, 128)]],
                rows_v.at[0], sem0,
            )

        pltpu.make_async_copy(
            table_hbm.at[idx_v.at[pl.ds((base + 1) * 128, 128)]],
            rows_v.at[1], sem1,
        ).wait()
        _process_chunk(rows_v, 1, base + 1, out_v)
        return 0

    lax.fori_loop(0, CHUNKS // 2, pair_body, 0)

    def scale_body(r, _):
        for g in range(SEGS):
            out_v[r, pl.ds(g * LANES, LANES)] = (
                out_v[r, pl.ds(g * LANES, LANES)] * (1.0 / L)
            )
        return 0
    lax.fori_loop(0, BPW, scale_body, 0, unroll=4)

    row_base = (wid & 15) * BPW
    col_base = (wid >> 4) * EMBED
    pltpu.sync_copy(
        out_v,
        out_hbm.at[pl.ds(row_base, BPW), pl.ds(col_base, EMBED)],
    )


def _head_body(emb_ref, w_ref, b_ref, out_ref):
    e = emb_ref[...]
    w = w_ref[...]
    bb = b_ref[...]
    dn = (((1,), (1,)), ((), ()))
    top = lax.dot_general(e[:, :EMBED], w, dn, preferred_element_type=jnp.float32)
    bot = lax.dot_general(e[:, EMBED:], w, dn, preferred_element_type=jnp.float32)
    out_ref[:HB, :] = 1.0 / (1.0 + jnp.exp(-(top + bb)))
    out_ref[HB:, :] = 1.0 / (1.0 + jnp.exp(-(bot + bb)))


_head = pl.pallas_call(
    _head_body,
    out_shape=jax.ShapeDtypeStruct((B, LABELS), jnp.float32),
)


def kernel(text, table, W, b):
    text1d = text.reshape(B * L)
    emb2 = _embed_bag(text1d, table)
    return _head(emb2, W, b.reshape(1, LABELS))


# speedup vs baseline: 1.9799x; 1.2637x over previous
import functools

import jax
import jax.numpy as jnp
from jax import lax
from jax.experimental import pallas as pl
from jax.experimental.pallas import tpu as pltpu
from jax.experimental.pallas import tpu_sc as plsc

EMBED = 64
LABELS = 4
B = 4096
L = 50

NC, NS, LANES = 2, 16, 16
NW = NC * NS
BPW = B // NW
IPW = BPW * L
CHUNKS = IPW // 128
SEGS = EMBED // LANES
HB = B // 2

_mesh = plsc.VectorSubcoreMesh(
    core_axis_name="c", subcore_axis_name="s", num_cores=NC, num_subcores=NS
)


def _process_chunk(rows_v, slot, c, out_v):
    def row_body(i, _):
        p = c * 128 + i
        el = (p * 5243) >> 18
        for g in range(SEGS):
            plsc.addupdate(
                out_v.at[el, pl.ds(g * LANES, LANES)],
                rows_v[slot, i, pl.ds(g * LANES, LANES)],
            )
        return 0
    lax.fori_loop(0, 128, row_body, 0, unroll=4)


@functools.partial(
    pl.kernel,
    out_type=jax.ShapeDtypeStruct((HB, 2 * EMBED), jnp.float32),
    mesh=_mesh,
    scratch_types=[
        pltpu.VMEM((IPW,), jnp.int32),
        pltpu.VMEM((2, 128, EMBED), jnp.float32),
        pltpu.VMEM((BPW, EMBED), jnp.float32),
        pltpu.SemaphoreType.DMA,
        pltpu.SemaphoreType.DMA,
    ],
    compiler_params=pltpu.CompilerParams(use_tc_tiling_on_sc=False),
)
def _embed_bag(text_hbm, table_hbm, out_hbm, idx_v, rows_v, out_v, sem0, sem1):
    wid = lax.axis_index("s") * NC + lax.axis_index("c")
    pltpu.sync_copy(text_hbm.at[pl.ds(wid * IPW, IPW)], idx_v)

    zero = jnp.zeros((LANES,), jnp.float32)
    def zero_body(r, _):
        for g in range(SEGS):
            out_v[r, pl.ds(g * LANES, LANES)] = zero
        return 0
    lax.fori_loop(0, BPW, zero_body, 0, unroll=4)

    pltpu.async_copy(
        table_hbm.at[idx_v.at[pl.ds(0, 128)]], rows_v.at[0], sem0
    )

    def pair_body(pp, _):
        base = 2 * pp
        pltpu.async_copy(
            table_hbm.at[idx_v.at[pl.ds((base + 1) * 128, 128)]],
            rows_v.at[1], sem1,
        )
        pltpu.make_async_copy(
            table_hbm.at[idx_v.at[pl.ds(base * 128, 128)]], rows_v.at[0], sem0
        ).wait()
        _process_chunk(rows_v, 0, base, out_v)

        @pl.when(base + 2 < CHUNKS)
        def _():
            pltpu.async_copy(
                table_hbm.at[idx_v.at[pl.ds((base + 2) * 128, 128)]],
                rows_v.at[0], sem0,
            )

        pltpu.make_async_copy(
            table_hbm.at[idx_v.at[pl.ds((base + 1) * 128, 128)]],
            rows_v.at[1], sem1,
        ).wait()
        _process_chunk(rows_v, 1, base + 1, out_v)
        return 0

    lax.fori_loop(0, CHUNKS // 2, pair_body, 0)

    def scale_body(r, _):
        for g in range(SEGS):
            out_v[r, pl.ds(g * LANES, LANES)] = (
                out_v[r, pl.ds(g * LANES, LANES)] * (1.0 / L)
            )
        return 0
    lax.fori_loop(0, BPW, scale_body, 0, unroll=4)

    row_base = (wid & 15) * BPW
    col_base = (wid >> 4) * EMBED
    pltpu.sync_copy(
        out_v,
        out_hbm.at[pl.ds(row_base, BPW), pl.ds(col_base, EMBED)],
    )


VCB = 1024
VHALF = 489 * VCB
VOCAB = 1000000


def _tpose_body(lo_ref, hi_ref, o_ref):
    o_ref[:, :EMBED] = jnp.transpose(lo_ref[...], (1, 0))
    o_ref[:, EMBED:] = jnp.transpose(hi_ref[...], (1, 0))


_repack_table = pl.pallas_call(
    _tpose_body,
    grid=(VHALF // VCB,),
    in_specs=[
        pl.BlockSpec((EMBED, VCB), lambda i: (0, i)),
        pl.BlockSpec((EMBED, VCB), lambda i: (0, jnp.minimum(489 + i, VOCAB // VCB))),
    ],
    out_specs=pl.BlockSpec((VCB, 2 * EMBED), lambda i: (i, 0)),
    out_shape=jax.ShapeDtypeStruct((VHALF, 2 * EMBED), jnp.float32),
)


def _head_body(emb_ref, w_ref, b_ref, out_ref):
    e = emb_ref[...]
    w = w_ref[...]
    bb = b_ref[...]
    dn = (((1,), (1,)), ((), ()))
    top = lax.dot_general(e[:, :EMBED], w, dn, preferred_element_type=jnp.float32)
    bot = lax.dot_general(e[:, EMBED:], w, dn, preferred_element_type=jnp.float32)
    out_ref[:HB, :] = 1.0 / (1.0 + jnp.exp(-(top + bb)))
    out_ref[HB:, :] = 1.0 / (1.0 + jnp.exp(-(bot + bb)))


_head = pl.pallas_call(
    _head_body,
    out_shape=jax.ShapeDtypeStruct((B, LABELS), jnp.float32),
)


def kernel(text, table, W, b):
    text1d = text.reshape(B * L)
    text1d = text1d * 2 - jnp.where(text1d >= VHALF, 2 * VHALF - 1, 0)
    tt = table.T
    table_rm = _repack_table(tt, tt)
    table_rm = table_rm.reshape(2 * VHALF, EMBED)
    emb2 = _embed_bag(text1d, table_rm)
    return _head(emb2, W, b.reshape(1, LABELS))


# speedup vs baseline: 2.9801x; 1.5052x over previous
import functools

import jax
import jax.numpy as jnp
from jax import lax
from jax.experimental import pallas as pl
from jax.experimental.pallas import tpu as pltpu
from jax.experimental.pallas import tpu_sc as plsc

EMBED = 64
LABELS = 4
B = 4096
L = 50

NC, NS, LANES = 2, 16, 16
NW = NC * NS
BPW = B // NW
IPW = BPW * L
CHUNKS = IPW // 128
SEGS = EMBED // LANES
HB = B // 2

_mesh = plsc.VectorSubcoreMesh(
    core_axis_name="c", subcore_axis_name="s", num_cores=NC, num_subcores=NS
)


def _process_chunk(rows_v, slot, c, out_v):
    def row_body(i, _):
        p = c * 128 + i
        el = (p * 5243) >> 18
        for g in range(SEGS):
            plsc.addupdate(
                out_v.at[el, pl.ds(g * LANES, LANES)],
                rows_v[slot, i, pl.ds(g * LANES, LANES)],
            )
        return 0
    lax.fori_loop(0, 128, row_body, 0, unroll=4)


@functools.partial(
    pl.kernel,
    out_type=jax.ShapeDtypeStruct((HB, 2 * EMBED), jnp.float32),
    mesh=_mesh,
    scratch_types=[
        pltpu.VMEM((IPW,), jnp.int32),
        pltpu.VMEM((2, 128, EMBED), jnp.float32),
        pltpu.VMEM((BPW, EMBED), jnp.float32),
        pltpu.SemaphoreType.DMA,
        pltpu.SemaphoreType.DMA,
    ],
    compiler_params=pltpu.CompilerParams(use_tc_tiling_on_sc=False),
)
def _embed_bag(text_hbm, table_hbm, out_hbm, idx_v, rows_v, out_v, sem0, sem1):
    wid = lax.axis_index("s") * NC + lax.axis_index("c")
    pltpu.sync_copy(text_hbm.at[pl.ds(wid * IPW, IPW)], idx_v)

    zero = jnp.zeros((LANES,), jnp.float32)
    def zero_body(r, _):
        for g in range(SEGS):
            out_v[r, pl.ds(g * LANES, LANES)] = zero
        return 0
    lax.fori_loop(0, BPW, zero_body, 0, unroll=4)

    pltpu.async_copy(
        table_hbm.at[idx_v.at[pl.ds(0, 128)]], rows_v.at[0], sem0
    )

    def pair_body(pp, _):
        base = 2 * pp
        pltpu.async_copy(
            table_hbm.at[idx_v.at[pl.ds((base + 1) * 128, 128)]],
            rows_v.at[1], sem1,
        )
        pltpu.make_async_copy(
            table_hbm.at[idx_v.at[pl.ds(base * 128, 128)]], rows_v.at[0], sem0
        ).wait()
        _process_chunk(rows_v, 0, base, out_v)

        @pl.when(base + 2 < CHUNKS)
        def _():
            pltpu.async_copy(
                table_hbm.at[idx_v.at[pl.ds((base + 2) * 128, 128)]],
                rows_v.at[0], sem0,
            )

        pltpu.make_async_copy(
            table_hbm.at[idx_v.at[pl.ds((base + 1) * 128, 128)]],
            rows_v.at[1], sem1,
        ).wait()
        _process_chunk(rows_v, 1, base + 1, out_v)
        return 0

    lax.fori_loop(0, CHUNKS // 2, pair_body, 0)

    def scale_body(r, _):
        for g in range(SEGS):
            out_v[r, pl.ds(g * LANES, LANES)] = (
                out_v[r, pl.ds(g * LANES, LANES)] * (1.0 / L)
            )
        return 0
    lax.fori_loop(0, BPW, scale_body, 0, unroll=4)

    row_base = (wid & 15) * BPW
    col_base = (wid >> 4) * EMBED
    pltpu.sync_copy(
        out_v,
        out_hbm.at[pl.ds(row_base, BPW), pl.ds(col_base, EMBED)],
    )


VCB = 4096
VSB = 123
VHALF = VSB * VCB
VOCAB = 1000000


def _tpose_body(lo_ref, hi_ref, o_ref):
    o_ref[:, :EMBED] = jnp.transpose(lo_ref[...], (1, 0))
    o_ref[:, EMBED:] = jnp.transpose(hi_ref[...], (1, 0))


_repack_table = pl.pallas_call(
    _tpose_body,
    grid=(VSB,),
    in_specs=[
        pl.BlockSpec((EMBED, VCB), lambda i: (0, i)),
        pl.BlockSpec((EMBED, VCB), lambda i: (0, jnp.minimum(VSB + i, VOCAB // VCB))),
    ],
    out_specs=pl.BlockSpec((VCB, 2 * EMBED), lambda i: (i, 0)),
    out_shape=jax.ShapeDtypeStruct((VHALF, 2 * EMBED), jnp.float32),
)


def _head_body(emb_ref, w_ref, b_ref, out_ref):
    e = emb_ref[...]
    w = w_ref[...]
    bb = b_ref[...]
    dn = (((1,), (1,)), ((), ()))
    top = lax.dot_general(e[:, :EMBED], w, dn, preferred_element_type=jnp.float32)
    bot = lax.dot_general(e[:, EMBED:], w, dn, preferred_element_type=jnp.float32)
    out_ref[:HB, :] = 1.0 / (1.0 + jnp.exp(-(top + bb)))
    out_ref[HB:, :] = 1.0 / (1.0 + jnp.exp(-(bot + bb)))


_head = pl.pallas_call(
    _head_body,
    out_shape=jax.ShapeDtypeStruct((B, LABELS), jnp.float32),
)


def kernel(text, table, W, b):
    text1d = text.reshape(B * L)
    text1d = text1d * 2 - jnp.where(text1d >= VHALF, 2 * VHALF - 1, 0)
    tt = table.T
    table_rm = _repack_table(tt, tt)
    table_rm = table_rm.reshape(2 * VHALF, EMBED)
    emb2 = _embed_bag(text1d, table_rm)
    return _head(emb2, W, b.reshape(1, LABELS))


# speedup vs baseline: 3.2519x; 1.0912x over previous
import functools

import jax
import jax.numpy as jnp
from jax import lax
from jax.experimental import pallas as pl
from jax.experimental.pallas import tpu as pltpu
from jax.experimental.pallas import tpu_sc as plsc

EMBED = 64
LABELS = 4
B = 4096
L = 50

NC, NS, LANES = 2, 16, 16
NW = NC * NS
BPW = B // NW
IPW = BPW * L
CHUNKS = IPW // 128
SEGS = EMBED // LANES
HB = B // 2

_mesh = plsc.VectorSubcoreMesh(
    core_axis_name="c", subcore_axis_name="s", num_cores=NC, num_subcores=NS
)


def _process_chunk(rows_v, slot, c, out_v):
    def row_body(i, _):
        p = c * 128 + i
        el = (p * 5243) >> 18
        for g in range(SEGS):
            plsc.addupdate(
                out_v.at[el, pl.ds(g * LANES, LANES)],
                rows_v[slot, i, pl.ds(g * LANES, LANES)],
            )
        return 0
    lax.fori_loop(0, 128, row_body, 0, unroll=4)


@functools.partial(
    pl.kernel,
    out_type=jax.ShapeDtypeStruct((HB, 2 * EMBED), jnp.float32),
    mesh=_mesh,
    scratch_types=[
        pltpu.VMEM((IPW,), jnp.int32),
        pltpu.VMEM((4, 128, EMBED), jnp.float32),
        pltpu.VMEM((BPW, EMBED), jnp.float32),
        pltpu.SemaphoreType.DMA,
        pltpu.SemaphoreType.DMA,
        pltpu.SemaphoreType.DMA,
        pltpu.SemaphoreType.DMA,
    ],
    compiler_params=pltpu.CompilerParams(use_tc_tiling_on_sc=False),
)
def _embed_bag(text_hbm, table_hbm, out_hbm, idx_v, rows_v, out_v,
               sem0, sem1, sem2, sem3):
    wid = lax.axis_index("s") * NC + lax.axis_index("c")
    sems = (sem0, sem1, sem2, sem3)
    pltpu.sync_copy(text_hbm.at[pl.ds(wid * IPW, IPW)], idx_v)

    def gather(j, s):
        return pltpu.async_copy(
            table_hbm.at[idx_v.at[pl.ds(j * 128, 128)]], rows_v.at[s], sems[s]
        )

    for s in range(4):
        gather(s, s)

    zero = jnp.zeros((LANES,), jnp.float32)
    def zero_body(r, _):
        for g in range(SEGS):
            out_v[r, pl.ds(g * LANES, LANES)] = zero
        return 0
    lax.fori_loop(0, BPW, zero_body, 0, unroll=4)

    def quad_body(q, _):
        for s in range(4):
            j = 4 * q + s

            @pl.when(j < CHUNKS)
            def _():
                pltpu.make_async_copy(
                    table_hbm.at[idx_v.at[pl.ds(j * 128, 128)]],
                    rows_v.at[s], sems[s],
                ).wait()
                _process_chunk(rows_v, s, j, out_v)

                @pl.when(j + 4 < CHUNKS)
                def _():
                    gather(j + 4, s)
        return 0

    lax.fori_loop(0, (CHUNKS + 3) // 4, quad_body, 0)

    def scale_body(r, _):
        for g in range(SEGS):
            out_v[r, pl.ds(g * LANES, LANES)] = (
                out_v[r, pl.ds(g * LANES, LANES)] * (1.0 / L)
            )
        return 0
    lax.fori_loop(0, BPW, scale_body, 0, unroll=4)

    row_base = (wid & 15) * BPW
    col_base = (wid >> 4) * EMBED
    pltpu.sync_copy(
        out_v,
        out_hbm.at[pl.ds(row_base, BPW), pl.ds(col_base, EMBED)],
    )


VCB = 8192
VSB = 62
VHALF = VSB * VCB
VOCAB = 1000000


def _tpose_body(lo_ref, hi_ref, o_ref):
    o_ref[:, :EMBED] = jnp.transpose(lo_ref[...], (1, 0))
    o_ref[:, EMBED:] = jnp.transpose(hi_ref[...], (1, 0))


_repack_table = pl.pallas_call(
    _tpose_body,
    grid=(VSB,),
    in_specs=[
        pl.BlockSpec((EMBED, VCB), lambda i: (0, i)),
        pl.BlockSpec((EMBED, VCB), lambda i: (0, jnp.minimum(VSB + i, VOCAB // VCB))),
    ],
    out_specs=pl.BlockSpec((VCB, 2 * EMBED), lambda i: (i, 0)),
    out_shape=jax.ShapeDtypeStruct((VHALF, 2 * EMBED), jnp.float32),
)


def _head_body(emb_ref, w_ref, b_ref, out_ref):
    e = emb_ref[...]
    w = w_ref[...]
    bb = b_ref[...]
    dn = (((1,), (1,)), ((), ()))
    top = lax.dot_general(e[:, :EMBED], w, dn, preferred_element_type=jnp.float32)
    bot = lax.dot_general(e[:, EMBED:], w, dn, preferred_element_type=jnp.float32)
    out_ref[:HB, :] = 1.0 / (1.0 + jnp.exp(-(top + bb)))
    out_ref[HB:, :] = 1.0 / (1.0 + jnp.exp(-(bot + bb)))


_head = pl.pallas_call(
    _head_body,
    out_shape=jax.ShapeDtypeStruct((B, LABELS), jnp.float32),
)


def kernel(text, table, W, b):
    text1d = text.reshape(B * L)
    text1d = text1d * 2 - jnp.where(text1d >= VHALF, 2 * VHALF - 1, 0)
    tt = table.T
    table_rm = _repack_table(tt, tt)
    table_rm = table_rm.reshape(2 * VHALF, EMBED)
    emb2 = _embed_bag(text1d, table_rm)
    return _head(emb2, W, b.reshape(1, LABELS))


# speedup vs baseline: 4.0170x; 1.2353x over previous
import functools

import jax
import jax.numpy as jnp
from jax import lax
from jax.experimental import pallas as pl
from jax.experimental.pallas import tpu as pltpu
from jax.experimental.pallas import tpu_sc as plsc

EMBED = 64
LABELS = 4
B = 4096
L = 50

NC, NS, LANES = 2, 16, 16
NW = NC * NS
BPW = B // NW
LP = 64
IPW = BPW * LP
CHUNKS = IPW // 128
SEGS = EMBED // LANES
HB = B // 2

_mesh = plsc.VectorSubcoreMesh(
    core_axis_name="c", subcore_axis_name="s", num_cores=NC, num_subcores=NS
)


def _process_chunk(rows_v, slot, c, out_v):
    for e2 in range(2):
        def red(l, acc):
            r = e2 * LP + l
            return tuple(
                acc[g] + rows_v[slot, r, pl.ds(g * LANES, LANES)]
                for g in range(SEGS)
            )
        init = tuple(jnp.zeros((LANES,), jnp.float32) for _ in range(SEGS))
        acc = lax.fori_loop(0, L, red, init, unroll=5)
        for g in range(SEGS):
            out_v[2 * c + e2, pl.ds(g * LANES, LANES)] = acc[g] * (1.0 / L)


@functools.partial(
    pl.kernel,
    out_type=jax.ShapeDtypeStruct((HB, 2 * EMBED), jnp.float32),
    mesh=_mesh,
    scratch_types=[
        pltpu.VMEM((IPW,), jnp.int32),
        pltpu.VMEM((4, 128, EMBED), jnp.float32),
        pltpu.VMEM((BPW, EMBED), jnp.float32),
        pltpu.SemaphoreType.DMA,
        pltpu.SemaphoreType.DMA,
        pltpu.SemaphoreType.DMA,
        pltpu.SemaphoreType.DMA,
    ],
    compiler_params=pltpu.CompilerParams(use_tc_tiling_on_sc=False),
)
def _embed_bag(text_hbm, table_hbm, out_hbm, idx_v, rows_v, out_v,
               sem0, sem1, sem2, sem3):
    wid = lax.axis_index("s") * NC + lax.axis_index("c")
    sems = (sem0, sem1, sem2, sem3)
    pltpu.sync_copy(text_hbm.at[pl.ds(wid * IPW, IPW)], idx_v)

    def gather(j, s):
        return pltpu.async_copy(
            table_hbm.at[idx_v.at[pl.ds(j * 128, 128)]], rows_v.at[s], sems[s]
        )

    for s in range(4):
        gather(s, s)

    def quad_body(q, _):
        for s in range(4):
            j = 4 * q + s

            @pl.when(j < CHUNKS)
            def _():
                pltpu.make_async_copy(
                    table_hbm.at[idx_v.at[pl.ds(j * 128, 128)]],
                    rows_v.at[s], sems[s],
                ).wait()
                _process_chunk(rows_v, s, j, out_v)

                @pl.when(j + 4 < CHUNKS)
                def _():
                    gather(j + 4, s)
        return 0

    lax.fori_loop(0, (CHUNKS + 3) // 4, quad_body, 0)

    row_base = (wid & 15) * BPW
    col_base = (wid >> 4) * EMBED
    pltpu.sync_copy(
        out_v,
        out_hbm.at[pl.ds(row_base, BPW), pl.ds(col_base, EMBED)],
    )


VCB = 8192
VSB = 62
VHALF = VSB * VCB
VOCAB = 1000000


def _tpose_body(lo_ref, hi_ref, o_ref):
    o_ref[:, :EMBED] = jnp.transpose(lo_ref[...], (1, 0))
    o_ref[:, EMBED:] = jnp.transpose(hi_ref[...], (1, 0))


_repack_table = pl.pallas_call(
    _tpose_body,
    grid=(VSB,),
    in_specs=[
        pl.BlockSpec((EMBED, VCB), lambda i: (0, i)),
        pl.BlockSpec((EMBED, VCB), lambda i: (0, jnp.minimum(VSB + i, VOCAB // VCB))),
    ],
    out_specs=pl.BlockSpec((VCB, 2 * EMBED), lambda i: (i, 0)),
    out_shape=jax.ShapeDtypeStruct((VHALF, 2 * EMBED), jnp.float32),
)


def _head_body(emb_ref, w_ref, b_ref, out_ref):
    e = emb_ref[...]
    w = w_ref[...]
    bb = b_ref[...]
    dn = (((1,), (1,)), ((), ()))
    top = lax.dot_general(e[:, :EMBED], w, dn, preferred_element_type=jnp.float32)
    bot = lax.dot_general(e[:, EMBED:], w, dn, preferred_element_type=jnp.float32)
    out_ref[:HB, :] = 1.0 / (1.0 + jnp.exp(-(top + bb)))
    out_ref[HB:, :] = 1.0 / (1.0 + jnp.exp(-(bot + bb)))


_head = pl.pallas_call(
    _head_body,
    out_shape=jax.ShapeDtypeStruct((B, LABELS), jnp.float32),
)


def kernel(text, table, W, b):
    textp = jnp.concatenate([text, text[:, :LP - L]], axis=1)
    text1d = textp.reshape(B * LP)
    text1d = text1d * 2 - jnp.where(text1d >= VHALF, 2 * VHALF - 1, 0)
    tt = table.T
    table_rm = _repack_table(tt, tt)
    table_rm = table_rm.reshape(2 * VHALF, EMBED)
    emb2 = _embed_bag(text1d, table_rm)
    return _head(emb2, W, b.reshape(1, LABELS))
